# Initial kernel scaffold; baseline (speedup 1.0000x reference)
import jax
import jax.numpy as jnp
from jax.experimental import pallas as pl


def kernel(loc):
    raise NotImplementedError("write your pallas kernel here")


# speedup vs baseline: 3.1921x; 3.1921x over previous
import functools

import jax
import jax.numpy as jnp
from jax import lax
from jax.experimental import pallas as pl
from jax.experimental.pallas import tpu as pltpu
from jax.experimental.pallas import tpu_sc as plsc

B = 4096
BX = 64
BY = 64
RW = 128
ROWS = B * BX * BY // RW
RPB = BX * BY // RW

NC = 2
NS = 16
NW = NC * NS
LPW = B // NW
RPW = ROWS // NW

ZROWS = 512
NZ = RPW // ZROWS

_mesh = plsc.VectorSubcoreMesh(
    core_axis_name="c", subcore_axis_name="s", num_cores=NC, num_subcores=NS
)


@functools.partial(
    pl.kernel,
    out_type=jax.ShapeDtypeStruct((ROWS, RW), jnp.int32),
    mesh=_mesh,
    compiler_params=pltpu.CompilerParams(needs_layout_passes=False),
    scratch_types=[
        pltpu.VMEM((LPW * 2,), jnp.int32),
        pltpu.VMEM((LPW,), jnp.int32),
        pltpu.VMEM((LPW, RW), jnp.int32),
        pltpu.VMEM((ZROWS, RW), jnp.int32),
        pltpu.SemaphoreType.DMA,
        pltpu.SemaphoreType.DMA,
    ],
)
def _onehot2d_sc(loc_hbm, out_hbm, loc_v, idx_v, rows_v, zero_v, zsem, ssem):
    wid = lax.axis_index("s") * NC + lax.axis_index("c")
    row_base = wid * RPW

    pltpu.sync_copy(loc_hbm.at[pl.ds(wid * LPW * 2, LPW * 2)], loc_v)

    zv = jnp.zeros((16,), jnp.int32)

    def _zero_zrow(i, carry):
        for cc in range(RW // 16):
            zero_v[i, pl.ds(cc * 16, 16)] = zv
        return carry

    lax.fori_loop(0, ZROWS, _zero_zrow, 0)

    def _zero_rrow(i, carry):
        for cc in range(RW // 16):
            rows_v[i, pl.ds(cc * 16, 16)] = zv
        return carry

    lax.fori_loop(0, LPW, _zero_rrow, 0)

    zcopies = [
        pltpu.async_copy(
            zero_v, out_hbm.at[pl.ds(row_base + i * ZROWS, ZROWS)], zsem
        )
        for i in range(NZ)
    ]

    iota = lax.iota(jnp.int32, 16)
    ones = jnp.full((16,), 1, jnp.int32)
    for g in range(LPW // 16):
        rv = iota + g * 16
        xv = plsc.load_gather(loc_v, [rv * 2])
        yv = plsc.load_gather(loc_v, [rv * 2 + 1])
        col = (xv & 1) * BY + yv
        plsc.store_scatter(rows_v, [rv, col], ones)
        idx_v[pl.ds(g * 16, 16)] = row_base + rv * RPB + lax.shift_right_logical(xv, 1)

    for cpy in zcopies:
        cpy.wait()

    pltpu.async_copy(rows_v, out_hbm.at[idx_v], ssem).wait()


def kernel(loc):
    out2d = _onehot2d_sc(loc.reshape(-1))
    return out2d.reshape(B, BX, BY, 1)
